# baseline (speedup 1.0000x reference)
import functools

import jax
import jax.numpy as jnp
from jax import lax
from jax.experimental import pallas as pl
from jax.experimental.pallas import tpu as pltpu
from jax.experimental.pallas import tpu_sc as plsc

MIDDLE = 30
FEATURES = 26
EMBED_DIM = 32
BATCH = 16384

FPAD = 28
FC = FPAD // 4
NC = 2
NS = 16
NW = NC * NS
BPW = BATCH // NW
BFP = BATCH * FPAD
RPW = BFP // NW
IDXW = 128
CHUNK = 512
KSUB = CHUNK // IDXW
NSTEP = RPW // CHUNK
NLISTS = NSTEP * KSUB
NVEC = NLISTS * (IDXW // 16)
OUT_ROWS = BFP // 4
ORPW = OUT_ROWS // NW
OCHUNK = CHUNK // 4

DENSE_BLK = 1024
XROWS_BLK = DENSE_BLK // 8 * (FC * 8)


def _build_lists(islab, idx_v, wid):
    lanes = lax.iota(jnp.int32, 16)

    @plsc.parallel_loop(0, NVEC, unroll=2)
    def bv(v):
        t_local = v // 14
        g = (v % 14) * 16 + lanes
        fc = g >> 5
        r2 = (g >> 2) & 7
        fr = g & 3
        f = jnp.where((fc == 6) & (fr >= 2), fr - 2, fc * 4 + fr)
        b_local = t_local * 8 + r2
        vals = plsc.load_gather(islab, [f, b_local])
        local = t_local * 224 + g
        lst = ((local >> 9) << 2) | (local & 3)
        plsc.store_scatter(idx_v, [lst, (local >> 2) & 127], vals)


NTILE = 580000 // 128
TTAIL = 580000 - NTILE * 128
TPW = -(-(NTILE + 1) // NW)


@functools.partial(
    pl.kernel,
    mesh=plsc.VectorSubcoreMesh(core_axis_name="c", subcore_axis_name="s"),
    out_type=jax.ShapeDtypeStruct((580000 * EMBED_DIM,), jnp.float32),
    scratch_types=[
        pltpu.VMEM((2, EMBED_DIM, 128), jnp.float32),
        pltpu.VMEM((2, EMBED_DIM * 128), jnp.float32),
        pltpu.VMEM((TTAIL * EMBED_DIM,), jnp.float32),
        pltpu.SemaphoreType.DMA,
        pltpu.SemaphoreType.DMA,
    ],
    compiler_params=pltpu.CompilerParams(
        use_tc_tiling_on_sc=True, needs_layout_passes=False
    ),
)
def _sc_retile(tt_hbm, tail_hbm, out_hbm, tbuf, obuf, tailbuf, isem, osem):
    wid = lax.axis_index("s") * NC + lax.axis_index("c")
    lanes = lax.iota(jnp.int32, 16)

    def in_dma(k, par, issue=True):
        t = wid + k * NW
        mk = pltpu.async_copy if issue else pltpu.make_async_copy
        return mk(
            tt_hbm.at[:, pl.ds(pl.multiple_of(t * 128, 128), 128)],
            tbuf.at[par],
            isem,
        )

    in_dma(0, 0)

    def step(k, par):
        t = wid + k * NW
        valid = t < NTILE

        @pl.when(wid + (k + 1) * NW < NTILE)
        def _():
            in_dma(k + 1, 1 - par)

        @pl.when(valid)
        def _():
            in_dma(k, par, issue=False).wait()

            @pl.when(k >= 2)
            def _():
                tp = wid + (k - 2) * NW
                pltpu.make_async_copy(
                    obuf.at[par],
                    out_hbm.at[
                        pl.ds(
                            pl.multiple_of(tp * (128 * EMBED_DIM), 4096),
                            128 * EMBED_DIM,
                        )
                    ],
                    osem,
                ).wait()

            @plsc.parallel_loop(0, 128, unroll=2)
            def tr(r):
                for h in range(2):
                    vals = plsc.load_gather(
                        tbuf.at[par], [lanes + h * 16, lanes * 0 + r]
                    )
                    obuf[par, pl.ds(r * EMBED_DIM + h * 16, 16)] = vals

            pltpu.async_copy(
                obuf.at[par],
                out_hbm.at[
                    pl.ds(
                        pl.multiple_of(t * (128 * EMBED_DIM), 4096),
                        128 * EMBED_DIM,
                    )
                ],
                osem,
            )

        return 1 - par

    lax.fori_loop(0, TPW, step, 0)

    for back in (2, 1):
        k = TPW - back
        t = wid + k * NW

        @pl.when(t < NTILE)
        def _():
            pltpu.make_async_copy(
                obuf.at[k % 2],
                out_hbm.at[
                    pl.ds(
                        pl.multiple_of(t * (128 * EMBED_DIM), 4096),
                        128 * EMBED_DIM,
                    )
                ],
                osem,
            ).wait()

    @pl.when(wid == 0)
    def _():
        pltpu.sync_copy(tail_hbm, tailbuf)
        pltpu.sync_copy(
            tailbuf,
            out_hbm.at[
                pl.ds(
                    pl.multiple_of(NTILE * 128 * EMBED_DIM, 4096),
                    TTAIL * EMBED_DIM,
                )
            ],
        )


def _gather_chunk(table_hbm, idx_v, rows_v, c, par, sem):
    return [
        pltpu.async_copy(
            table_hbm.at[idx_v.at[c * KSUB + j]],
            rows_v.at[par, pl.ds(j * IDXW, IDXW)],
            sem,
        )
        for j in range(KSUB)
    ]


def _repack_chunk(rows_v, obuf, par):
    @plsc.parallel_loop(0, OCHUNK, unroll=2)
    def rp(r):
        vals = [
            rows_v[par, j * IDXW + r, pl.ds(h * 16, 16)]
            for j in range(KSUB)
            for h in range(2)
        ]
        for k, v in enumerate(vals):
            obuf[par, r, pl.ds(k * 16, 16)] = v


@functools.partial(
    pl.kernel,
    mesh=plsc.VectorSubcoreMesh(core_axis_name="c", subcore_axis_name="s"),
    out_type=jax.ShapeDtypeStruct((OUT_ROWS, 128), jnp.float32),
    scratch_types=[
        pltpu.VMEM((FEATURES, BPW), jnp.int32),
        pltpu.VMEM((NLISTS, IDXW), jnp.int32),
        pltpu.VMEM((2, CHUNK, EMBED_DIM), jnp.float32),
        pltpu.VMEM((2, OCHUNK, 128), jnp.float32),
        pltpu.SemaphoreType.DMA,
        pltpu.SemaphoreType.DMA,
    ],
    compiler_params=pltpu.CompilerParams(
        use_tc_tiling_on_sc=False, needs_layout_passes=False
    ),
)
def _sc_gather(idxt_hbm, table_hbm, out_hbm, islab, idx_v, rows_v, obuf,
               gsem, osem):
    wid = lax.axis_index("s") * NC + lax.axis_index("c")
    obase = wid * ORPW
    pltpu.sync_copy(idxt_hbm.at[:, pl.ds(wid * BPW, BPW)], islab)
    _build_lists(islab, idx_v, wid)

    _gather_chunk(table_hbm, idx_v, rows_v, 0, 0, gsem)

    def step(c, par):
        @pl.when(c + 1 < NSTEP)
        def _():
            _gather_chunk(table_hbm, idx_v, rows_v, c + 1, 1 - par, gsem)

        for j in range(KSUB):
            pltpu.make_async_copy(
                table_hbm.at[idx_v.at[c * KSUB + j]],
                rows_v.at[par, pl.ds(j * IDXW, IDXW)],
                gsem,
            ).wait()

        @pl.when(c >= 2)
        def _():
            off_prev = pl.multiple_of(obase + (c - 2) * OCHUNK, OCHUNK)
            pltpu.make_async_copy(
                obuf.at[par], out_hbm.at[pl.ds(off_prev, OCHUNK)], osem
            ).wait()

        _repack_chunk(rows_v, obuf, par)

        off = pl.multiple_of(obase + c * OCHUNK, OCHUNK)
        pltpu.async_copy(obuf.at[par], out_hbm.at[pl.ds(off, OCHUNK)], osem)
        return 1 - par

    lax.fori_loop(0, NSTEP, step, 0)

    for back in (2, 1):
        c = NSTEP - back
        off = pl.multiple_of(obase + c * OCHUNK, OCHUNK)
        pltpu.make_async_copy(
            obuf.at[c % 2], out_hbm.at[pl.ds(off, OCHUNK)], osem
        ).wait()


def _dense_body(x_ref, w_ref, b_ref, o_ref):
    x = x_ref[...].reshape(DENSE_BLK // 8, FC, 8, 128)
    acc = jnp.zeros((DENSE_BLK, MIDDLE), dtype=jnp.float32)
    for j in range(FC):
        xj = x[:, j].reshape(DENSE_BLK, 128)
        acc += jnp.dot(
            xj,
            w_ref[pl.ds(j * 128, 128), :],
            preferred_element_type=jnp.float32,
        )
    o_ref[...] = jnp.maximum(acc + b_ref[...], 0.0)


def kernel(input, table, W, b):
    idxt = input.astype(jnp.int32).T
    tail = table[NTILE * 128 :, :].reshape(TTAIL * EMBED_DIM)
    table_lin = _sc_retile(table.T, tail).reshape(580000, EMBED_DIM)
    gathered = _sc_gather(idxt, table_lin)

    w_pad = jnp.pad(W, ((0, FC * 128 - FEATURES * EMBED_DIM), (0, 0)))
    out = pl.pallas_call(
        _dense_body,
        grid=(BATCH // DENSE_BLK,),
        in_specs=[
            pl.BlockSpec((XROWS_BLK, 128), lambda i: (i, 0)),
            pl.BlockSpec((FC * 128, MIDDLE), lambda i: (0, 0)),
            pl.BlockSpec((1, MIDDLE), lambda i: (0, 0)),
        ],
        out_specs=pl.BlockSpec((DENSE_BLK, MIDDLE), lambda i: (i, 0)),
        out_shape=jax.ShapeDtypeStruct((BATCH, MIDDLE), jnp.float32),
    )(gathered, w_pad, b.reshape(1, MIDDLE))
    return out


# speedup vs baseline: 1.4773x; 1.4773x over previous
import functools

import jax
import jax.numpy as jnp
from jax import lax
from jax.experimental import pallas as pl
from jax.experimental.pallas import tpu as pltpu
from jax.experimental.pallas import tpu_sc as plsc

MIDDLE = 30
FEATURES = 26
EMBED_DIM = 32
BATCH = 16384

FPAD = 28
FC = FPAD // 4
NC = 2
NS = 16
NW = NC * NS
BPW = BATCH // NW
BFP = BATCH * FPAD
RPW = BFP // NW
IDXW = 128
CHUNK = 512
KSUB = CHUNK // IDXW
NSTEP = RPW // CHUNK
NLISTS = NSTEP * KSUB
NVEC = NLISTS * (IDXW // 16)
OUT_ROWS = BFP // 4
ORPW = OUT_ROWS // NW
OCHUNK = CHUNK // 4

DENSE_BLK = 1024
XROWS_BLK = DENSE_BLK // 8 * (FC * 8)


def _build_lists(islab, idx_v, wid):
    lanes = lax.iota(jnp.int32, 16)

    @plsc.parallel_loop(0, NVEC, unroll=2)
    def bv(v):
        t_local = v // 14
        g = (v % 14) * 16 + lanes
        fc = g >> 5
        r2 = (g >> 2) & 7
        fr = g & 3
        f = jnp.where((fc == 6) & (fr >= 2), fr - 2, fc * 4 + fr)
        b_local = t_local * 8 + r2
        vals = plsc.load_gather(islab, [f, b_local])
        local = t_local * 224 + g
        lst = ((local >> 9) << 2) | (local & 3)
        plsc.store_scatter(idx_v, [lst, (local >> 2) & 127], vals)


NTILE = 580000 // 128
TTAIL = 580000 - NTILE * 128
TPW = -(-(NTILE + 1) // NW)


@functools.partial(
    pl.kernel,
    mesh=plsc.VectorSubcoreMesh(core_axis_name="c", subcore_axis_name="s"),
    out_type=jax.ShapeDtypeStruct((580000 * EMBED_DIM,), jnp.float32),
    scratch_types=[
        pltpu.VMEM((2, EMBED_DIM, 128), jnp.float32),
        pltpu.VMEM((2, EMBED_DIM * 128), jnp.float32),
        pltpu.VMEM((128 * 33,), jnp.float32),
        pltpu.VMEM((TTAIL * EMBED_DIM,), jnp.float32),
        pltpu.SemaphoreType.DMA,
        pltpu.SemaphoreType.DMA,
    ],
    compiler_params=pltpu.CompilerParams(
        use_tc_tiling_on_sc=True, needs_layout_passes=False
    ),
)
def _sc_retile(tt_hbm, tail_hbm, out_hbm, tbuf, obuf, stage, tailbuf,
               isem, osem):
    wid = lax.axis_index("s") * NC + lax.axis_index("c")
    lanes = lax.iota(jnp.int32, 16)

    def in_dma(k, par, issue=True):
        t = wid + k * NW
        mk = pltpu.async_copy if issue else pltpu.make_async_copy
        return mk(
            tt_hbm.at[:, pl.ds(pl.multiple_of(t * 128, 128), 128)],
            tbuf.at[par],
            isem,
        )

    in_dma(0, 0)

    def step(k, par):
        t = wid + k * NW
        valid = t < NTILE

        @pl.when(wid + (k + 1) * NW < NTILE)
        def _():
            in_dma(k + 1, 1 - par)

        @pl.when(valid)
        def _():
            in_dma(k, par, issue=False).wait()

            @pl.when(k >= 2)
            def _():
                tp = wid + (k - 2) * NW
                pltpu.make_async_copy(
                    obuf.at[par],
                    out_hbm.at[
                        pl.ds(
                            pl.multiple_of(tp * (128 * EMBED_DIM), 4096),
                            128 * EMBED_DIM,
                        )
                    ],
                    osem,
                ).wait()

            @plsc.parallel_loop(0, EMBED_DIM * 8, unroll=2)
            def tr(v):
                d = v >> 3
                r0 = (v & 7) * 16
                vals = tbuf[par, d, pl.ds(r0, 16)]
                plsc.store_scatter(stage, [(r0 + lanes) * 33 + d], vals)

            @plsc.parallel_loop(0, 128, unroll=2)
            def tr2(r):
                for h in range(2):
                    obuf[par, pl.ds(r * EMBED_DIM + h * 16, 16)] = stage[
                        pl.ds(r * 33 + h * 16, 16)
                    ]

            pltpu.async_copy(
                obuf.at[par],
                out_hbm.at[
                    pl.ds(
                        pl.multiple_of(t * (128 * EMBED_DIM), 4096),
                        128 * EMBED_DIM,
                    )
                ],
                osem,
            )

        return 1 - par

    lax.fori_loop(0, TPW, step, 0)

    for back in (2, 1):
        k = TPW - back
        t = wid + k * NW

        @pl.when(t < NTILE)
        def _():
            pltpu.make_async_copy(
                obuf.at[k % 2],
                out_hbm.at[
                    pl.ds(
                        pl.multiple_of(t * (128 * EMBED_DIM), 4096),
                        128 * EMBED_DIM,
                    )
                ],
                osem,
            ).wait()

    @pl.when(wid == 0)
    def _():
        pltpu.sync_copy(tail_hbm, tailbuf)
        pltpu.sync_copy(
            tailbuf,
            out_hbm.at[
                pl.ds(
                    pl.multiple_of(NTILE * 128 * EMBED_DIM, 4096),
                    TTAIL * EMBED_DIM,
                )
            ],
        )


def _gather_chunk(table_hbm, idx_v, rows_v, c, par, sem):
    return [
        pltpu.async_copy(
            table_hbm.at[idx_v.at[c * KSUB + j]],
            rows_v.at[par, pl.ds(j * IDXW, IDXW)],
            sem,
        )
        for j in range(KSUB)
    ]


def _repack_chunk(rows_v, obuf, par):
    @plsc.parallel_loop(0, OCHUNK, unroll=2)
    def rp(r):
        vals = [
            rows_v[par, j * IDXW + r, pl.ds(h * 16, 16)]
            for j in range(KSUB)
            for h in range(2)
        ]
        for k, v in enumerate(vals):
            obuf[par, r, pl.ds(k * 16, 16)] = v


@functools.partial(
    pl.kernel,
    mesh=plsc.VectorSubcoreMesh(core_axis_name="c", subcore_axis_name="s"),
    out_type=jax.ShapeDtypeStruct((OUT_ROWS, 128), jnp.float32),
    scratch_types=[
        pltpu.VMEM((FEATURES, BPW), jnp.int32),
        pltpu.VMEM((NLISTS, IDXW), jnp.int32),
        pltpu.VMEM((2, CHUNK, EMBED_DIM), jnp.float32),
        pltpu.VMEM((2, OCHUNK, 128), jnp.float32),
        pltpu.SemaphoreType.DMA,
        pltpu.SemaphoreType.DMA,
    ],
    compiler_params=pltpu.CompilerParams(
        use_tc_tiling_on_sc=False, needs_layout_passes=False
    ),
)
def _sc_gather(idxt_hbm, table_hbm, out_hbm, islab, idx_v, rows_v, obuf,
               gsem, osem):
    wid = lax.axis_index("s") * NC + lax.axis_index("c")
    obase = wid * ORPW
    pltpu.sync_copy(idxt_hbm.at[:, pl.ds(wid * BPW, BPW)], islab)
    _build_lists(islab, idx_v, wid)

    _gather_chunk(table_hbm, idx_v, rows_v, 0, 0, gsem)

    def step(c, par):
        @pl.when(c + 1 < NSTEP)
        def _():
            _gather_chunk(table_hbm, idx_v, rows_v, c + 1, 1 - par, gsem)

        for j in range(KSUB):
            pltpu.make_async_copy(
                table_hbm.at[idx_v.at[c * KSUB + j]],
                rows_v.at[par, pl.ds(j * IDXW, IDXW)],
                gsem,
            ).wait()

        @pl.when(c >= 2)
        def _():
            off_prev = pl.multiple_of(obase + (c - 2) * OCHUNK, OCHUNK)
            pltpu.make_async_copy(
                obuf.at[par], out_hbm.at[pl.ds(off_prev, OCHUNK)], osem
            ).wait()

        _repack_chunk(rows_v, obuf, par)

        off = pl.multiple_of(obase + c * OCHUNK, OCHUNK)
        pltpu.async_copy(obuf.at[par], out_hbm.at[pl.ds(off, OCHUNK)], osem)
        return 1 - par

    lax.fori_loop(0, NSTEP, step, 0)

    for back in (2, 1):
        c = NSTEP - back
        off = pl.multiple_of(obase + c * OCHUNK, OCHUNK)
        pltpu.make_async_copy(
            obuf.at[c % 2], out_hbm.at[pl.ds(off, OCHUNK)], osem
        ).wait()


def _dense_body(x_ref, w_ref, b_ref, o_ref):
    x = x_ref[...].reshape(DENSE_BLK // 8, FC, 8, 128)
    acc = jnp.zeros((DENSE_BLK, MIDDLE), dtype=jnp.float32)
    for j in range(FC):
        xj = x[:, j].reshape(DENSE_BLK, 128)
        acc += jnp.dot(
            xj,
            w_ref[pl.ds(j * 128, 128), :],
            preferred_element_type=jnp.float32,
        )
    o_ref[...] = jnp.maximum(acc + b_ref[...], 0.0)


def kernel(input, table, W, b):
    idxt = input.astype(jnp.int32).T
    tail = table[NTILE * 128 :, :].reshape(TTAIL * EMBED_DIM)
    table_lin = _sc_retile(table.T, tail).reshape(580000, EMBED_DIM)
    gathered = _sc_gather(idxt, table_lin)

    w_pad = jnp.pad(W, ((0, FC * 128 - FEATURES * EMBED_DIM), (0, 0)))
    out = pl.pallas_call(
        _dense_body,
        grid=(BATCH // DENSE_BLK,),
        in_specs=[
            pl.BlockSpec((XROWS_BLK, 128), lambda i: (i, 0)),
            pl.BlockSpec((FC * 128, MIDDLE), lambda i: (0, 0)),
            pl.BlockSpec((1, MIDDLE), lambda i: (0, 0)),
        ],
        out_specs=pl.BlockSpec((DENSE_BLK, MIDDLE), lambda i: (i, 0)),
        out_shape=jax.ShapeDtypeStruct((BATCH, MIDDLE), jnp.float32),
    )(gathered, w_pad, b.reshape(1, MIDDLE))
    return out
